# baseline (speedup 1.0000x reference)
import jax
import jax.numpy as jnp
from jax import lax
from jax.experimental import pallas as pl
from jax.experimental.pallas import tpu as pltpu
from jax.experimental.pallas import tpu_sc as plsc

_INTERVAL_SIZE = 8
_I_FRAME_POSITION = 3
_THRESHOLD = 50.0
_PATCH_NUM = 196
_NUM_FRAMES = 32
_IMG_START = 64
_IMG_LEN = _PATCH_NUM * _NUM_FRAMES
_ROWS_PER_INTERVAL = _PATCH_NUM * _INTERVAL_SIZE
_SEQ_LEN = 6400
_D_MODEL = 4096
_PRE = 128
_LANES = 16
_CHUNK = 224
_IPRE_ROWS = 200


def _lanesum(v):
    for sh in (8, 4, 2, 1):
        idx = lax.iota(jnp.int32, 16) ^ sh
        v = v + jnp.take(v, idx)
    return v


def _sc_body(x_hbm, out_hbm, xpre, ipre, xw8, iw8, flags, bulk_sem):
    cid = lax.axis_index("c")
    sid = lax.axis_index("s")
    w = sid * 2 + cid
    is_img = w < 28
    interval = w // 7
    img_r0 = _IMG_START + _ROWS_PER_INTERVAL * interval + _CHUNK * (w % 7)
    aux_r0 = jnp.where(w < 30, 32 * (w - 28), _IMG_START + _IMG_LEN + 32 * (w - 30))
    ipre_r0 = _IMG_START + _PATCH_NUM * (interval * _INTERVAL_SIZE + _I_FRAME_POSITION) - 4

    @pl.when(is_img)
    def _image_worker():
        r0 = pl.multiple_of(img_r0, 8)
        bulk = pltpu.make_async_copy(
            x_hbm.at[pl.ds(r0, _CHUNK)], out_hbm.at[pl.ds(r0, _CHUNK)], bulk_sem
        )
        bulk.start()

        pltpu.sync_copy(x_hbm.at[pl.ds(r0, _CHUNK), pl.ds(0, _PRE)], xpre)
        pltpu.sync_copy(
            x_hbm.at[pl.ds(pl.multiple_of(ipre_r0, 8), _IPRE_ROWS), pl.ds(0, _PRE)],
            ipre,
        )

        rel0 = r0 - _IMG_START - _ROWS_PER_INTERVAL * interval

        def _decide(r, carry):
            p = (rel0 + r) % _PATCH_NUM
            acc = jnp.zeros((_LANES,), jnp.float32)
            for k in range(_PRE // _LANES):
                xv = xpre[r, pl.ds(k * _LANES, _LANES)]
                iv = ipre[4 + p, pl.ds(k * _LANES, _LANES)]
                acc = acc + jnp.abs(xv - iv)
            partial = _lanesum(acc)[0]
            flags[r] = jnp.where(partial < _THRESHOLD, 1, 0)
            return carry

        lax.fori_loop(0, _CHUNK, _decide, 0)

        bulk.wait()

        def _resolve(r, carry):
            @pl.when(flags[r] == 1)
            def _full_check():
                gr = r0 + r
                p = (rel0 + r) % _PATCH_NUM
                gra = pl.multiple_of((gr // 8) * 8, 8)
                r8 = gr - gra
                ig = ipre_r0 + 4 + p
                iga = pl.multiple_of((ig // 8) * 8, 8)
                ig8 = ig - iga
                pltpu.sync_copy(x_hbm.at[pl.ds(gra, 8)], xw8)
                pltpu.sync_copy(x_hbm.at[pl.ds(iga, 8)], iw8)

                def _chunk_sad(k, a):
                    xv = xw8[r8, pl.ds(k * _LANES, _LANES)]
                    iv = iw8[ig8, pl.ds(k * _LANES, _LANES)]
                    return a + jnp.abs(xv - iv)

                acc2 = lax.fori_loop(
                    0, _D_MODEL // _LANES, _chunk_sad, jnp.zeros((_LANES,), jnp.float32)
                )
                sad = _lanesum(acc2)[0]

                @pl.when(sad < _THRESHOLD)
                def _scatter():
                    pltpu.sync_copy(out_hbm.at[pl.ds(gra, 8)], xw8)
                    for k in range(_D_MODEL // _LANES):
                        xw8[r8, pl.ds(k * _LANES, _LANES)] = iw8[
                            ig8, pl.ds(k * _LANES, _LANES)
                        ]
                    pltpu.sync_copy(xw8, out_hbm.at[pl.ds(gra, 8)])

            return carry

        lax.fori_loop(0, _CHUNK, _resolve, 0)

    @pl.when(jnp.logical_not(is_img))
    def _aux_worker():
        r0 = pl.multiple_of(aux_r0, 8)
        aux = pltpu.make_async_copy(
            x_hbm.at[pl.ds(r0, 32)], out_hbm.at[pl.ds(r0, 32)], bulk_sem
        )
        aux.start()
        aux.wait()


def kernel(hidden_states):
    x = hidden_states[0]
    mesh = plsc.VectorSubcoreMesh(core_axis_name="c", subcore_axis_name="s")
    out = pl.kernel(
        _sc_body,
        out_type=jax.ShapeDtypeStruct((_SEQ_LEN, _D_MODEL), jnp.float32),
        mesh=mesh,
        scratch_types=[
            pltpu.VMEM((_CHUNK, _PRE), jnp.float32),
            pltpu.VMEM((_IPRE_ROWS, _PRE), jnp.float32),
            pltpu.VMEM((8, _D_MODEL), jnp.float32),
            pltpu.VMEM((8, _D_MODEL), jnp.float32),
            pltpu.SMEM((_CHUNK,), jnp.int32),
            pltpu.SemaphoreType.DMA,
        ],
    )(x)
    return out[None]


# speedup vs baseline: 1.2877x; 1.2877x over previous
import jax
import jax.numpy as jnp
from jax import lax
from jax.experimental import pallas as pl
from jax.experimental.pallas import tpu as pltpu
from jax.experimental.pallas import tpu_sc as plsc

_INTERVAL_SIZE = 8
_I_FRAME_POSITION = 3
_THRESHOLD = 50.0
_PATCH_NUM = 196
_NUM_FRAMES = 32
_IMG_START = 64
_IMG_LEN = _PATCH_NUM * _NUM_FRAMES
_ROWS_PER_INTERVAL = _PATCH_NUM * _INTERVAL_SIZE
_SEQ_LEN = 6400
_D_MODEL = 4096
_PRE = 128
_LANES = 16
_CHUNK = 224
_IPRE_ROWS = 200


def _lanesum(v):
    for sh in (8, 4, 2, 1):
        idx = lax.iota(jnp.int32, 16) ^ sh
        v = v + jnp.take(v, idx)
    return v


def _sc_body(x_hbm, out_hbm, xpre, ipre, xw8, iw8, flags, bulk_sem):
    cid = lax.axis_index("c")
    sid = lax.axis_index("s")
    w = sid * 2 + cid
    is_img = w < 28
    interval = w // 7
    img_r0 = _IMG_START + _ROWS_PER_INTERVAL * interval + _CHUNK * (w % 7)
    aux_r0 = jnp.where(w < 30, 32 * (w - 28), _IMG_START + _IMG_LEN + 32 * (w - 30))
    ipre_r0 = _IMG_START + _PATCH_NUM * (interval * _INTERVAL_SIZE + _I_FRAME_POSITION) - 4

    @pl.when(is_img)
    def _image_worker():
        r0 = pl.multiple_of(img_r0, 8)
        bulk = pltpu.make_async_copy(
            x_hbm.at[pl.ds(r0, _CHUNK)], out_hbm.at[pl.ds(r0, _CHUNK)], bulk_sem
        )
        bulk.start()

        _EXP_A = True
        if not _EXP_A:
            pltpu.sync_copy(x_hbm.at[pl.ds(r0, _CHUNK), pl.ds(0, _PRE)], xpre)
            pltpu.sync_copy(
                x_hbm.at[pl.ds(pl.multiple_of(ipre_r0, 8), _IPRE_ROWS), pl.ds(0, _PRE)],
                ipre,
            )

        rel0 = r0 - _IMG_START - _ROWS_PER_INTERVAL * interval

        def _decide(r, carry):
            p = (rel0 + r) % _PATCH_NUM
            acc = jnp.zeros((_LANES,), jnp.float32)
            for k in range(_PRE // _LANES):
                xv = xpre[r, pl.ds(k * _LANES, _LANES)]
                iv = ipre[4 + p, pl.ds(k * _LANES, _LANES)]
                acc = acc + jnp.abs(xv - iv)
            partial = _lanesum(acc)[0]
            flags[r] = jnp.where(partial < _THRESHOLD, 1, 0)
            return carry

        if not _EXP_A:
            lax.fori_loop(0, _CHUNK, _decide, 0)

        bulk.wait()

        def _resolve(r, carry):
            @pl.when(flags[r] == 1)
            def _full_check():
                gr = r0 + r
                p = (rel0 + r) % _PATCH_NUM
                gra = pl.multiple_of((gr // 8) * 8, 8)
                r8 = gr - gra
                ig = ipre_r0 + 4 + p
                iga = pl.multiple_of((ig // 8) * 8, 8)
                ig8 = ig - iga
                pltpu.sync_copy(x_hbm.at[pl.ds(gra, 8)], xw8)
                pltpu.sync_copy(x_hbm.at[pl.ds(iga, 8)], iw8)

                def _chunk_sad(k, a):
                    xv = xw8[r8, pl.ds(k * _LANES, _LANES)]
                    iv = iw8[ig8, pl.ds(k * _LANES, _LANES)]
                    return a + jnp.abs(xv - iv)

                acc2 = lax.fori_loop(
                    0, _D_MODEL // _LANES, _chunk_sad, jnp.zeros((_LANES,), jnp.float32)
                )
                sad = _lanesum(acc2)[0]

                @pl.when(sad < _THRESHOLD)
                def _scatter():
                    pltpu.sync_copy(out_hbm.at[pl.ds(gra, 8)], xw8)
                    for k in range(_D_MODEL // _LANES):
                        xw8[r8, pl.ds(k * _LANES, _LANES)] = iw8[
                            ig8, pl.ds(k * _LANES, _LANES)
                        ]
                    pltpu.sync_copy(xw8, out_hbm.at[pl.ds(gra, 8)])

            return carry

        if not _EXP_A:
            lax.fori_loop(0, _CHUNK, _resolve, 0)

    @pl.when(jnp.logical_not(is_img))
    def _aux_worker():
        r0 = pl.multiple_of(aux_r0, 8)
        aux = pltpu.make_async_copy(
            x_hbm.at[pl.ds(r0, 32)], out_hbm.at[pl.ds(r0, 32)], bulk_sem
        )
        aux.start()
        aux.wait()


def kernel(hidden_states):
    x = hidden_states[0]
    mesh = plsc.VectorSubcoreMesh(core_axis_name="c", subcore_axis_name="s")
    out = pl.kernel(
        _sc_body,
        out_type=jax.ShapeDtypeStruct((_SEQ_LEN, _D_MODEL), jnp.float32),
        mesh=mesh,
        scratch_types=[
            pltpu.VMEM((_CHUNK, _PRE), jnp.float32),
            pltpu.VMEM((_IPRE_ROWS, _PRE), jnp.float32),
            pltpu.VMEM((8, _D_MODEL), jnp.float32),
            pltpu.VMEM((8, _D_MODEL), jnp.float32),
            pltpu.SMEM((_CHUNK,), jnp.int32),
            pltpu.SemaphoreType.DMA,
        ],
    )(x)
    return out[None]


# speedup vs baseline: 42.4254x; 32.9461x over previous
import jax
import jax.numpy as jnp
from jax import lax
from jax.experimental import pallas as pl
from jax.experimental.pallas import tpu as pltpu
from jax.experimental.pallas import tpu_sc as plsc

_INTERVAL_SIZE = 8
_I_FRAME_POSITION = 3
_THRESHOLD = 50.0
_PATCH_NUM = 196
_NUM_FRAMES = 32
_IMG_START = 64
_IMG_LEN = _PATCH_NUM * _NUM_FRAMES
_ROWS_PER_INTERVAL = _PATCH_NUM * _INTERVAL_SIZE
_SEQ_LEN = 6400
_D_MODEL = 4096
_PRE = 128
_LANES = 16
_CHUNK = 224
_CROWS = 8
_NCHUNKS = _CHUNK // _CROWS
_IPRE_ROWS = 200
_AUX_ROWS = 32


def _lanesum(v):
    for sh in (8, 4, 2, 1):
        idx = lax.iota(jnp.int32, 16) ^ sh
        v = v + jnp.take(v, idx)
    return v


def _sc_body(x_hbm, out_hbm, ipre, cb0, cb1, iw, sem_r, sem_w, sem_i):
    cid = lax.axis_index("c")
    sid = lax.axis_index("s")
    w = sid * 2 + cid
    is_img = w < 28
    interval = w // 7
    img_r0 = _IMG_START + _ROWS_PER_INTERVAL * interval + _CHUNK * (w % 7)
    aux_r0 = jnp.where(w < 30, 32 * (w - 28), _IMG_START + _IMG_LEN + 32 * (w - 30))
    ipre_r0 = _IMG_START + _PATCH_NUM * (interval * _INTERVAL_SIZE + _I_FRAME_POSITION) - 4

    @pl.when(is_img)
    def _image_worker():
        r0 = pl.multiple_of(img_r0, 8)
        rel0 = r0 - _IMG_START - _ROWS_PER_INTERVAL * interval

        def _read(c, buf):
            return pltpu.make_async_copy(
                x_hbm.at[pl.ds(pl.multiple_of(r0 + _CROWS * c, 8), _CROWS)], buf, sem_r
            )

        def _write(c, buf):
            return pltpu.make_async_copy(
                buf, out_hbm.at[pl.ds(pl.multiple_of(r0 + _CROWS * c, 8), _CROWS)], sem_w
            )

        pltpu.sync_copy(
            x_hbm.at[pl.ds(pl.multiple_of(ipre_r0, 8), _IPRE_ROWS), pl.ds(0, _PRE)],
            ipre,
        )
        _read(0, cb0).start()

        def _process(c, cur):
            def _row(r8, carry):
                rel = rel0 + _CROWS * c + r8
                fin = rel // _PATCH_NUM
                p = rel % _PATCH_NUM
                acc = jnp.zeros((_LANES,), jnp.float32)
                for k in range(_PRE // _LANES):
                    xv = cur[r8, pl.ds(k * _LANES, _LANES)]
                    iv = ipre[4 + p, pl.ds(k * _LANES, _LANES)]
                    acc = acc + jnp.abs(xv - iv)
                partial = _lanesum(acc)[0]
                undecided = jnp.logical_and(
                    partial < _THRESHOLD, fin != _I_FRAME_POSITION
                )

                @pl.when(undecided)
                def _full_check():
                    ig = ipre_r0 + 4 + p
                    iga = pl.multiple_of((ig // 8) * 8, 8)
                    ig8 = ig - iga
                    pltpu.make_async_copy(
                        x_hbm.at[pl.ds(iga, 8)], iw, sem_i
                    ).start()
                    pltpu.make_async_copy(
                        x_hbm.at[pl.ds(iga, 8)], iw, sem_i
                    ).wait()

                    def _chunk_sad(k, a):
                        xv = cur[r8, pl.ds(k * _LANES, _LANES)]
                        iv = iw[ig8, pl.ds(k * _LANES, _LANES)]
                        return a + jnp.abs(xv - iv)

                    acc2 = lax.fori_loop(
                        0,
                        _D_MODEL // _LANES,
                        _chunk_sad,
                        jnp.zeros((_LANES,), jnp.float32),
                    )
                    sad = _lanesum(acc2)[0]

                    @pl.when(sad < _THRESHOLD)
                    def _substitute():
                        def _copy_chunk(k, cc):
                            cur[r8, pl.ds(k * _LANES, _LANES)] = iw[
                                ig8, pl.ds(k * _LANES, _LANES)
                            ]
                            return cc

                        lax.fori_loop(0, _D_MODEL // _LANES, _copy_chunk, 0)

                return carry

            lax.fori_loop(0, _CROWS, _row, 0)

        def _do_chunk(c, cur, other):
            _read(c, cur).wait()

            @pl.when(c > 0)
            def _drain_prev_write():
                _write(c - 1, other).wait()

            @pl.when(c + 1 < _NCHUNKS)
            def _prefetch():
                _read(c + 1, other).start()

            _process(c, cur)
            _write(c, cur).start()

        def _chunk_step(c, carry):
            @pl.when(c % 2 == 0)
            def _even():
                _do_chunk(c, cb0, cb1)

            @pl.when(c % 2 == 1)
            def _odd():
                _do_chunk(c, cb1, cb0)

            return carry

        lax.fori_loop(0, _NCHUNKS, _chunk_step, 0)
        _write(_NCHUNKS - 1, cb1).wait()

    @pl.when(jnp.logical_not(is_img))
    def _aux_worker():
        r0 = pl.multiple_of(aux_r0, 8)
        for c in range(_AUX_ROWS // _CROWS):
            pltpu.sync_copy(x_hbm.at[pl.ds(pl.multiple_of(r0 + _CROWS * c, 8), _CROWS)], cb0)
            pltpu.sync_copy(cb0, out_hbm.at[pl.ds(pl.multiple_of(r0 + _CROWS * c, 8), _CROWS)])


def kernel(hidden_states):
    x = hidden_states[0]
    mesh = plsc.VectorSubcoreMesh(core_axis_name="c", subcore_axis_name="s")
    out = pl.kernel(
        _sc_body,
        out_type=jax.ShapeDtypeStruct((_SEQ_LEN, _D_MODEL), jnp.float32),
        mesh=mesh,
        scratch_types=[
            pltpu.VMEM((_IPRE_ROWS, _PRE), jnp.float32),
            pltpu.VMEM((_CROWS, _D_MODEL), jnp.float32),
            pltpu.VMEM((_CROWS, _D_MODEL), jnp.float32),
            pltpu.VMEM((8, _D_MODEL), jnp.float32),
            pltpu.SemaphoreType.DMA,
            pltpu.SemaphoreType.DMA,
            pltpu.SemaphoreType.DMA,
        ],
    )(x)
    return out[None]
